# baseline (speedup 1.0000x reference)
import jax
import jax.numpy as jnp
from jax.experimental import pallas as pl
from jax.experimental.pallas import tpu as pltpu

_R, _C = 2048, 4096
_CHUNK = 64
_NCHUNK = _R // _CHUNK


def _copy_body(in_ref, out_ref, *scratch):
    bufs = scratch[:_NCHUNK]
    isems = scratch[_NCHUNK:2 * _NCHUNK]
    osems = scratch[2 * _NCHUNK:]
    in2d = in_ref.reshape(_R, _C)

    def rows(c):
        return pl.ds(c * _CHUNK, _CHUNK)

    ins = []
    for c in range(_NCHUNK):
        cp = pltpu.make_async_copy(in2d.at[rows(c)], bufs[c], isems[c])
        cp.start()
        ins.append(cp)
    outs = []
    for c in range(_NCHUNK):
        ins[c].wait()
        cp = pltpu.make_async_copy(bufs[c], out_ref.at[rows(c)], osems[c])
        cp.start()
        outs.append(cp)
    for cp in outs:
        cp.wait()


def kernel(free_values):
    x = free_values.reshape(_R * _C // 128, 128)
    return pl.pallas_call(
        _copy_body,
        in_specs=[pl.BlockSpec(memory_space=pl.ANY)],
        out_specs=pl.BlockSpec(memory_space=pl.ANY),
        out_shape=jax.ShapeDtypeStruct((_R, _C), jnp.float32),
        scratch_shapes=(
            [pltpu.VMEM((_CHUNK, _C), jnp.float32) for _ in range(_NCHUNK)]
            + [pltpu.SemaphoreType.DMA for _ in range(2 * _NCHUNK)]
        ),
    )(x)


# speedup vs baseline: 1.0374x; 1.0374x over previous
import jax
import jax.numpy as jnp
from jax.experimental import pallas as pl
from jax.experimental.pallas import tpu as pltpu

_R, _C = 2048, 4096
_CHUNK = 256
_NCHUNK = _R // _CHUNK


def _copy_body(in_ref, out_ref, *scratch):
    bufs = scratch[:_NCHUNK]
    isems = scratch[_NCHUNK:2 * _NCHUNK]
    osems = scratch[2 * _NCHUNK:]
    in2d = in_ref.reshape(_R, _C)

    def rows(c):
        return pl.ds(c * _CHUNK, _CHUNK)

    ins = []
    for c in range(_NCHUNK):
        cp = pltpu.make_async_copy(in2d.at[rows(c)], bufs[c], isems[c])
        cp.start()
        ins.append(cp)
    outs = []
    for c in range(_NCHUNK):
        ins[c].wait()
        cp = pltpu.make_async_copy(bufs[c], out_ref.at[rows(c)], osems[c])
        cp.start()
        outs.append(cp)
    for cp in outs:
        cp.wait()


def kernel(free_values):
    x = free_values.reshape(_R * _C // 128, 128)
    return pl.pallas_call(
        _copy_body,
        in_specs=[pl.BlockSpec(memory_space=pl.ANY)],
        out_specs=pl.BlockSpec(memory_space=pl.ANY),
        out_shape=jax.ShapeDtypeStruct((_R, _C), jnp.float32),
        scratch_shapes=(
            [pltpu.VMEM((_CHUNK, _C), jnp.float32) for _ in range(_NCHUNK)]
            + [pltpu.SemaphoreType.DMA for _ in range(2 * _NCHUNK)]
        ),
    )(x)


# speedup vs baseline: 1.0560x; 1.0180x over previous
import jax
import jax.numpy as jnp
from jax.experimental import pallas as pl
from jax.experimental.pallas import tpu as pltpu

_R, _C = 2048, 4096
_CHUNK = 512
_NCHUNK = _R // _CHUNK


def _copy_body(in_ref, out_ref, *scratch):
    bufs = scratch[:_NCHUNK]
    isems = scratch[_NCHUNK:2 * _NCHUNK]
    osems = scratch[2 * _NCHUNK:]
    in2d = in_ref.reshape(_R, _C)

    def rows(c):
        return pl.ds(c * _CHUNK, _CHUNK)

    ins = []
    for c in range(_NCHUNK):
        cp = pltpu.make_async_copy(in2d.at[rows(c)], bufs[c], isems[c])
        cp.start()
        ins.append(cp)
    outs = []
    for c in range(_NCHUNK):
        ins[c].wait()
        cp = pltpu.make_async_copy(bufs[c], out_ref.at[rows(c)], osems[c])
        cp.start()
        outs.append(cp)
    for cp in outs:
        cp.wait()


def kernel(free_values):
    x = free_values.reshape(_R * _C // 128, 128)
    return pl.pallas_call(
        _copy_body,
        in_specs=[pl.BlockSpec(memory_space=pl.ANY)],
        out_specs=pl.BlockSpec(memory_space=pl.ANY),
        out_shape=jax.ShapeDtypeStruct((_R, _C), jnp.float32),
        scratch_shapes=(
            [pltpu.VMEM((_CHUNK, _C), jnp.float32) for _ in range(_NCHUNK)]
            + [pltpu.SemaphoreType.DMA for _ in range(2 * _NCHUNK)]
        ),
    )(x)


# speedup vs baseline: 1.0600x; 1.0038x over previous
import jax
import jax.numpy as jnp
from jax.experimental import pallas as pl
from jax.experimental.pallas import tpu as pltpu

_R, _C = 2048, 4096
_CHUNK = 1024
_NCHUNK = _R // _CHUNK


def _copy_body(in_ref, out_ref, *scratch):
    bufs = scratch[:_NCHUNK]
    isems = scratch[_NCHUNK:2 * _NCHUNK]
    osems = scratch[2 * _NCHUNK:]
    in2d = in_ref.reshape(_R, _C)

    def rows(c):
        return pl.ds(c * _CHUNK, _CHUNK)

    ins = []
    for c in range(_NCHUNK):
        cp = pltpu.make_async_copy(in2d.at[rows(c)], bufs[c], isems[c])
        cp.start()
        ins.append(cp)
    outs = []
    for c in range(_NCHUNK):
        ins[c].wait()
        cp = pltpu.make_async_copy(bufs[c], out_ref.at[rows(c)], osems[c])
        cp.start()
        outs.append(cp)
    for cp in outs:
        cp.wait()


def kernel(free_values):
    x = free_values.reshape(_R * _C // 128, 128)
    return pl.pallas_call(
        _copy_body,
        in_specs=[pl.BlockSpec(memory_space=pl.ANY)],
        out_specs=pl.BlockSpec(memory_space=pl.ANY),
        out_shape=jax.ShapeDtypeStruct((_R, _C), jnp.float32),
        scratch_shapes=(
            [pltpu.VMEM((_CHUNK, _C), jnp.float32) for _ in range(_NCHUNK)]
            + [pltpu.SemaphoreType.DMA for _ in range(2 * _NCHUNK)]
        ),
    )(x)
